# Initial kernel scaffold; baseline (speedup 1.0000x reference)
import jax
import jax.numpy as jnp
from jax.experimental import pallas as pl


def kernel(x, edge_index, ln_g, ln_b, W1, b1, bn1_g, bn1_b, bn1_m, bn1_v, W2, b2, bn2_g, bn2_b, bn2_m, bn2_v, Wc1, bc1, lnc_g, lnc_b, Wc2, bc2):
    raise NotImplementedError("write your pallas kernel here")


# speedup vs baseline: 12.3443x; 12.3443x over previous
import functools

import jax
import jax.numpy as jnp
from jax import lax
from jax.experimental import pallas as pl
from jax.experimental.pallas import tpu as pltpu
from jax.experimental.pallas import tpu_sc as plsc

N = 10000
E = 160000
D = 256
H = 256
H2 = 128
H4 = 64
C = 8
EPS = 1e-5

NC = 2
NS = 16
NP = 10240
RPS = NP // NS

AGG_CHUNK = 80
AGG_CHUNKS = E // NS // AGG_CHUNK

ES_CHUNK = 40
ES_CHUNKS = E // (NC * NS) // ES_CHUNK

_mesh = lambda: plsc.VectorSubcoreMesh(core_axis_name="c", subcore_axis_name="s")


@functools.partial(
    pl.kernel,
    mesh=_mesh(),
    out_type=[
        jax.ShapeDtypeStruct((NP,), jnp.float32),
        jax.ShapeDtypeStruct((NP,), jnp.float32),
    ],
    scratch_types=[
        pltpu.VMEM_SHARED((NP,), jnp.float32),
        pltpu.VMEM((ES_CHUNKS, ES_CHUNK), jnp.int32),
        pltpu.VMEM((ES_CHUNK,), jnp.float32),
    ],
)
def _deg_kernel(dst_hbm, ones_hbm, zeros_hbm, out0, out1, acc, idxs, ones_v):
    c = lax.axis_index("c")
    s = lax.axis_index("s")
    pltpu.sync_copy(ones_hbm, ones_v)
    pltpu.sync_copy(dst_hbm.at[c, s], idxs)
    pltpu.sync_copy(zeros_hbm, acc.at[pl.ds(s * RPS, RPS)])
    plsc.subcore_barrier()

    @pl.loop(0, ES_CHUNKS)
    def _(j):
        pltpu.sync_copy(ones_v, acc.at[idxs.at[j]], add=True)

    plsc.subcore_barrier()

    @pl.when(c == 0)
    def _():
        pltpu.sync_copy(acc.at[pl.ds(s * RPS, RPS)], out0.at[pl.ds(s * RPS, RPS)])

    @pl.when(c == 1)
    def _():
        pltpu.sync_copy(acc.at[pl.ds(s * RPS, RPS)], out1.at[pl.ds(s * RPS, RPS)])


HC1 = H // 2


@functools.partial(
    pl.kernel,
    mesh=_mesh(),
    out_type=[
        jax.ShapeDtypeStruct((NP, HC1), jnp.float32),
        jax.ShapeDtypeStruct((NP, HC1), jnp.float32),
    ],
    scratch_types=[
        pltpu.VMEM_SHARED((NP, HC1), jnp.float32),
        pltpu.VMEM((AGG_CHUNKS, AGG_CHUNK), jnp.int32),
        pltpu.VMEM((AGG_CHUNKS, AGG_CHUNK), jnp.int32),
        pltpu.VMEM((AGG_CHUNK, HC1), jnp.float32),
        pltpu.SemaphoreType.DMA,
    ],
)
def _agg_conv1(u0, u1, src_hbm, dst_hbm, out0, out1, acc, sidx, didx, rows, sem):
    c = lax.axis_index("c")
    s = lax.axis_index("s")
    pltpu.sync_copy(src_hbm.at[s], sidx)
    pltpu.sync_copy(dst_hbm.at[s], didx)

    def run(u, out):
        pltpu.sync_copy(u.at[pl.ds(s * RPS, RPS)], acc.at[pl.ds(s * RPS, RPS)])
        plsc.subcore_barrier()

        @pl.loop(0, AGG_CHUNKS)
        def _(j):
            pltpu.async_copy(u.at[sidx.at[j]], rows, sem).wait()
            pltpu.sync_copy(rows, acc.at[didx.at[j]], add=True)

        plsc.subcore_barrier()
        pltpu.sync_copy(acc.at[pl.ds(s * RPS, RPS)], out.at[pl.ds(s * RPS, RPS)])

    @pl.when(c == 0)
    def _():
        run(u0, out0)

    @pl.when(c == 1)
    def _():
        run(u1, out1)


@functools.partial(
    pl.kernel,
    mesh=_mesh(),
    out_type=[
        jax.ShapeDtypeStruct((NP, H2), jnp.float32),
        jax.ShapeDtypeStruct((NP, H2), jnp.float32),
    ],
    scratch_types=[
        pltpu.VMEM_SHARED((NP, H2), jnp.float32),
        pltpu.VMEM((ES_CHUNKS, ES_CHUNK), jnp.int32),
        pltpu.VMEM((ES_CHUNKS, ES_CHUNK), jnp.int32),
        pltpu.VMEM((ES_CHUNK, H2), jnp.float32),
        pltpu.SemaphoreType.DMA,
    ],
)
def _agg_conv2(u, src_hbm, dst_hbm, zeros_hbm, out0, out1,
               acc, sidx, didx, rows, sem):
    c = lax.axis_index("c")
    s = lax.axis_index("s")
    pltpu.sync_copy(src_hbm.at[c, s], sidx)
    pltpu.sync_copy(dst_hbm.at[c, s], didx)

    @pl.when(c == 0)
    def _():
        pltpu.sync_copy(u.at[pl.ds(s * RPS, RPS)], acc.at[pl.ds(s * RPS, RPS)])

    @pl.when(c == 1)
    def _():
        pltpu.sync_copy(zeros_hbm, acc.at[pl.ds(s * RPS, RPS)])

    plsc.subcore_barrier()

    @pl.loop(0, ES_CHUNKS)
    def _(j):
        pltpu.async_copy(u.at[sidx.at[j]], rows, sem).wait()
        pltpu.sync_copy(rows, acc.at[didx.at[j]], add=True)

    plsc.subcore_barrier()

    @pl.when(c == 0)
    def _():
        pltpu.sync_copy(acc.at[pl.ds(s * RPS, RPS)], out0.at[pl.ds(s * RPS, RPS)])

    @pl.when(c == 1)
    def _():
        pltpu.sync_copy(acc.at[pl.ds(s * RPS, RPS)], out1.at[pl.ds(s * RPS, RPS)])


BN_ROWS = 2000


def _dinv_block(d0, d1):
    deg = d0 + d1 + 1.0
    return lax.rsqrt(deg)


def _tc1_body(x_ref, g_ref, b_ref, w_ref, d0_ref, d1_ref, u0_ref, u1_ref):
    x = x_ref[...]
    mu = jnp.mean(x, axis=1, keepdims=True)
    var = jnp.mean((x - mu) * (x - mu), axis=1, keepdims=True)
    xn = (x - mu) * lax.rsqrt(var + EPS) * g_ref[...] + b_ref[...]
    h = jnp.dot(xn, w_ref[...], preferred_element_type=jnp.float32)
    u = h * _dinv_block(d0_ref[...], d1_ref[...])
    u0_ref[...] = u[:, :HC1]
    u1_ref[...] = u[:, HC1:]


def _tc2_body(a0_ref, a1_ref, d0_ref, d1_ref, b1_ref, g_ref, b_ref, m_ref,
              v_ref, w_ref, u_ref):
    dinv = _dinv_block(d0_ref[...], d1_ref[...])
    t = jnp.concatenate([a0_ref[...], a1_ref[...]], axis=1) * dinv + b1_ref[...]
    t = (t - m_ref[...]) * lax.rsqrt(v_ref[...] + EPS) * g_ref[...] + b_ref[...]
    t = jnp.maximum(t, 0.0)
    h = jnp.dot(t, w_ref[...], preferred_element_type=jnp.float32)
    u_ref[...] = h * dinv


def _tc3_body(a0_ref, a1_ref, d0_ref, d1_ref, b2_ref, g_ref, b_ref, m_ref,
              v_ref, wc1_ref, bc1_ref, lg_ref, lb_ref, wc2_ref, bc2_ref, o_ref):
    dinv = _dinv_block(d0_ref[...], d1_ref[...])
    t = (a0_ref[...] + a1_ref[...]) * dinv + b2_ref[...]
    t = (t - m_ref[...]) * lax.rsqrt(v_ref[...] + EPS) * g_ref[...] + b_ref[...]
    t = jnp.maximum(t, 0.0)
    h = jnp.dot(t, wc1_ref[...], preferred_element_type=jnp.float32) + bc1_ref[...]
    mu = jnp.mean(h, axis=1, keepdims=True)
    var = jnp.mean((h - mu) * (h - mu), axis=1, keepdims=True)
    h = (h - mu) * lax.rsqrt(var + EPS) * lg_ref[...] + lb_ref[...]
    h = jnp.maximum(h, 0.0)
    o_ref[...] = jnp.dot(h, wc2_ref[...], preferred_element_type=jnp.float32) + bc2_ref[...]


def _row_spec(width):
    return pl.BlockSpec((BN_ROWS, width), lambda i: (i, 0))


def _full_spec(shape):
    return pl.BlockSpec(shape, lambda i: tuple(0 for _ in shape))


def kernel(x, edge_index, ln_g, ln_b, W1, b1, bn1_g, bn1_b, bn1_m, bn1_v,
           W2, b2, bn2_g, bn2_b, bn2_m, bn2_v, Wc1, bc1, lnc_g, lnc_b,
           Wc2, bc2):
    src = edge_index[0]
    dst = edge_index[1]
    src_agg = src.reshape(NS, AGG_CHUNKS, AGG_CHUNK)
    dst_agg = dst.reshape(NS, AGG_CHUNKS, AGG_CHUNK)
    src_es = src.reshape(NC, NS, ES_CHUNKS, ES_CHUNK)
    dst_es = dst.reshape(NC, NS, ES_CHUNKS, ES_CHUNK)
    ones_blk = jnp.ones((ES_CHUNK,), jnp.float32)
    zeros_1d = jnp.zeros((RPS,), jnp.float32)
    zeros_2d = jnp.zeros((RPS, H2), jnp.float32)

    deg0, deg1 = _deg_kernel(dst_es, ones_blk, zeros_1d)
    deg0 = deg0.reshape(NP, 1)
    deg1 = deg1.reshape(NP, 1)

    grid = (N // BN_ROWS,)
    r1 = lambda: _row_spec(1)

    u0, u1 = pl.pallas_call(
        _tc1_body,
        grid=grid,
        in_specs=[
            _row_spec(D),
            _full_spec((1, D)),
            _full_spec((1, D)),
            _full_spec((D, H)),
            r1(),
            r1(),
        ],
        out_specs=[_row_spec(HC1), _row_spec(HC1)],
        out_shape=[
            jax.ShapeDtypeStruct((NP, HC1), jnp.float32),
            jax.ShapeDtypeStruct((NP, HC1), jnp.float32),
        ],
    )(x, ln_g.reshape(1, D), ln_b.reshape(1, D), W1, deg0, deg1)

    s0, s1 = _agg_conv1(u0, u1, src_agg, dst_agg)

    u2, = pl.pallas_call(
        _tc2_body,
        grid=grid,
        in_specs=[
            _row_spec(HC1),
            _row_spec(HC1),
            r1(),
            r1(),
            _full_spec((1, H)),
            _full_spec((1, H)),
            _full_spec((1, H)),
            _full_spec((1, H)),
            _full_spec((1, H)),
            _full_spec((H, H2)),
        ],
        out_specs=[_row_spec(H2)],
        out_shape=[jax.ShapeDtypeStruct((NP, H2), jnp.float32)],
    )(s0, s1, deg0, deg1, b1.reshape(1, H), bn1_g.reshape(1, H),
      bn1_b.reshape(1, H), bn1_m.reshape(1, H), bn1_v.reshape(1, H), W2)

    s20, s21 = _agg_conv2(u2, src_es, dst_es, zeros_2d)

    out = pl.pallas_call(
        _tc3_body,
        grid=grid,
        in_specs=[
            _row_spec(H2),
            _row_spec(H2),
            r1(),
            r1(),
            _full_spec((1, H2)),
            _full_spec((1, H2)),
            _full_spec((1, H2)),
            _full_spec((1, H2)),
            _full_spec((1, H2)),
            _full_spec((H2, H4)),
            _full_spec((1, H4)),
            _full_spec((1, H4)),
            _full_spec((1, H4)),
            _full_spec((H4, C)),
            _full_spec((1, C)),
        ],
        out_specs=_row_spec(C),
        out_shape=jax.ShapeDtypeStruct((N, C), jnp.float32),
    )(s20, s21, deg0, deg1, b2.reshape(1, H2), bn2_g.reshape(1, H2),
      bn2_b.reshape(1, H2), bn2_m.reshape(1, H2), bn2_v.reshape(1, H2),
      Wc1, bc1.reshape(1, H4), lnc_g.reshape(1, H4), lnc_b.reshape(1, H4),
      Wc2, bc2.reshape(1, C))

    return out
